# baseline (speedup 1.0000x reference)
import functools

import jax
import jax.numpy as jnp
from jax import lax
from jax.experimental import pallas as pl
from jax.experimental.pallas import tpu as pltpu
from jax.experimental.pallas import tpu_sc as plsc

_NC, _NS = 2, 16
_NW = _NC * _NS
_B = 16384
_D = 16
_BPW = _B // _NW
_BLK = 16
_NBLK = _BPW // _BLK


def _build_gather():
    mesh = plsc.VectorSubcoreMesh(core_axis_name="c", subcore_axis_name="s")

    @functools.partial(
        pl.kernel,
        mesh=mesh,
        out_type=jax.ShapeDtypeStruct((_D, _B), jnp.float32),
        scratch_types=[
            pltpu.VMEM((_BPW // 16, 16), jnp.int32),
            pltpu.VMEM((_D, _BPW), jnp.float32),
        ]
        + [pltpu.VMEM((_D, 128), jnp.float32) for _ in range(_BLK)]
        + [pltpu.SemaphoreType.DMA],
        compiler_params=pltpu.CompilerParams(needs_layout_passes=False),
    )
    def gather_kernel(tableT_hbm, idx_hbm, outT_hbm, idx_v, cols_v, *rest):
        bufs = rest[:_BLK]
        sem = rest[_BLK]
        wid = lax.axis_index("s") * _NC + lax.axis_index("c")
        base = wid * _BPW
        pltpu.sync_copy(idx_hbm.at[wid], idx_v)
        lanes = lax.iota(jnp.int32, 16)

        def block(b, carry):
            v = idx_v[b]
            cps = []
            for j in range(_BLK):
                i = v[j]
                c0 = pl.multiple_of(lax.bitwise_and(i, jnp.int32(-128)), 128)
                cps.append(
                    pltpu.async_copy(
                        tableT_hbm.at[:, pl.ds(c0, 128)],
                        bufs[j],
                        sem,
                    )
                )
            for j in range(_BLK):
                cps[j].wait()
                i = v[j]
                col = lax.bitwise_and(i, jnp.int32(127))
                vals = plsc.load_gather(
                    bufs[j], [lanes, jnp.full((16,), col, jnp.int32)]
                )
                plsc.store_scatter(
                    cols_v,
                    [lanes, jnp.full((16,), b * _BLK + j, jnp.int32)],
                    vals,
                )
            return carry

        lax.fori_loop(0, _NBLK, block, 0)
        pltpu.sync_copy(cols_v, outT_hbm.at[:, pl.ds(base, _BPW)])

    return gather_kernel


_gather = _build_gather()


def kernel(meta_data, indices):
    idx = indices.astype(jnp.int32).reshape(_NW, _BPW // 16, 16)
    return _gather(meta_data.T, idx).T


# speedup vs baseline: 1.0005x; 1.0005x over previous
import functools

import jax
import jax.numpy as jnp
from jax import lax
from jax.experimental import pallas as pl
from jax.experimental.pallas import tpu as pltpu
from jax.experimental.pallas import tpu_sc as plsc

_NC, _NS = 2, 16
_NW = _NC * _NS
_B = 16384
_D = 16
_BPW = _B // _NW
_BLK = 16
_NBLK = _BPW // _BLK


def _build_gather():
    mesh = plsc.VectorSubcoreMesh(core_axis_name="c", subcore_axis_name="s")

    @functools.partial(
        pl.kernel,
        mesh=mesh,
        out_type=jax.ShapeDtypeStruct((_D, _B), jnp.float32),
        scratch_types=[
            pltpu.VMEM((_BPW // 128, 128), jnp.int32),
            pltpu.VMEM((_D, _BPW), jnp.float32),
        ]
        + [pltpu.VMEM((_D, 128), jnp.float32) for _ in range(_BLK)]
        + [pltpu.SemaphoreType.DMA],
        compiler_params=pltpu.CompilerParams(needs_layout_passes=False),
    )
    def gather_kernel(tableT_hbm, idx_hbm, outT_hbm, idx_v, cols_v, *rest):
        bufs = rest[:_BLK]
        sem = rest[_BLK]
        wid = lax.axis_index("s") * _NC + lax.axis_index("c")
        base = wid * _BPW
        pltpu.sync_copy(idx_hbm.at[pl.ds(wid * (_BPW // 128), _BPW // 128)], idx_v)
        lanes = lax.iota(jnp.int32, 16)

        def block(b, carry):
            v = idx_v[lax.shift_right_logical(b, 3),
                      pl.ds(lax.bitwise_and(b, 7) * 16, 16)]
            cps = []
            for j in range(_BLK):
                i = v[j]
                c0 = pl.multiple_of(lax.bitwise_and(i, jnp.int32(-128)), 128)
                cps.append(
                    pltpu.async_copy(
                        tableT_hbm.at[:, pl.ds(c0, 128)],
                        bufs[j],
                        sem,
                    )
                )
            for j in range(_BLK):
                cps[j].wait()
                i = v[j]
                col = lax.bitwise_and(i, jnp.int32(127))
                vals = plsc.load_gather(
                    bufs[j], [lanes, jnp.full((16,), col, jnp.int32)]
                )
                plsc.store_scatter(
                    cols_v,
                    [lanes, jnp.full((16,), b * _BLK + j, jnp.int32)],
                    vals,
                )
            return carry

        lax.fori_loop(0, _NBLK, block, 0)
        pltpu.sync_copy(cols_v, outT_hbm.at[:, pl.ds(base, _BPW)])

    return gather_kernel


_gather = _build_gather()


def kernel(meta_data, indices):
    idx = indices.astype(jnp.int32).reshape(_B // 128, 128)
    return _gather(meta_data.T, idx).T


# speedup vs baseline: 1.1358x; 1.1352x over previous
import functools

import jax
import jax.numpy as jnp
from jax import lax
from jax.experimental import pallas as pl
from jax.experimental.pallas import tpu as pltpu
from jax.experimental.pallas import tpu_sc as plsc

_NC, _NS = 2, 16
_NW = _NC * _NS
_B = 16384
_D = 16
_BPW = _B // _NW
_BLK = 32
_NBLK = _BPW // _BLK


def _build_gather():
    mesh = plsc.VectorSubcoreMesh(core_axis_name="c", subcore_axis_name="s")

    @functools.partial(
        pl.kernel,
        mesh=mesh,
        out_type=jax.ShapeDtypeStruct((_D, _B), jnp.float32),
        scratch_types=[
            pltpu.VMEM((_BPW // 128, 128), jnp.int32),
            pltpu.VMEM((_D, _BPW), jnp.float32),
        ]
        + [pltpu.VMEM((_D, 128), jnp.float32) for _ in range(_BLK)]
        + [pltpu.SemaphoreType.DMA],
        compiler_params=pltpu.CompilerParams(needs_layout_passes=False),
    )
    def gather_kernel(tableT_hbm, idx_hbm, outT_hbm, idx_v, cols_v, *rest):
        bufs = rest[:_BLK]
        sem = rest[_BLK]
        wid = lax.axis_index("s") * _NC + lax.axis_index("c")
        base = wid * _BPW
        pltpu.sync_copy(idx_hbm.at[pl.ds(wid * (_BPW // 128), _BPW // 128)], idx_v)
        lanes = lax.iota(jnp.int32, 16)

        def block(b, carry):
            m0 = b * _BLK
            vs = [
                idx_v[lax.div(m0 + g * 16, 128),
                      pl.ds(lax.rem(m0 + g * 16, 128), 16)]
                for g in range(_BLK // 16)
            ]
            cps = []
            for j in range(_BLK):
                i = vs[j // 16][j % 16]
                c0 = pl.multiple_of(lax.bitwise_and(i, jnp.int32(-128)), 128)
                cps.append(
                    pltpu.async_copy(
                        tableT_hbm.at[:, pl.ds(c0, 128)],
                        bufs[j],
                        sem,
                    )
                )
            for j in range(_BLK):
                cps[j].wait()
                i = vs[j // 16][j % 16]
                col = lax.bitwise_and(i, jnp.int32(127))
                vals = plsc.load_gather(
                    bufs[j], [lanes, jnp.full((16,), col, jnp.int32)]
                )
                plsc.store_scatter(
                    cols_v,
                    [lanes, jnp.full((16,), m0 + j, jnp.int32)],
                    vals,
                )
            return carry

        lax.fori_loop(0, _NBLK, block, 0)
        pltpu.sync_copy(cols_v, outT_hbm.at[:, pl.ds(base, _BPW)])

    return gather_kernel


_gather = _build_gather()


def kernel(meta_data, indices):
    idx = indices.astype(jnp.int32).reshape(_B // 128, 128)
    return _gather(meta_data.T, idx).T
